# Initial kernel scaffold; baseline (speedup 1.0000x reference)
import jax
import jax.numpy as jnp
from jax.experimental import pallas as pl


def kernel(x, edge_index, W, b):
    raise NotImplementedError("write your pallas kernel here")


# speedup vs baseline: 8.4770x; 8.4770x over previous
import functools

import jax
import jax.numpy as jnp
from jax import lax
from jax.experimental import pallas as pl
from jax.experimental.pallas import tpu as pltpu
from jax.experimental.pallas import tpu_sc as plsc

_NC = 2
_NS = 16
_NW = _NC * _NS
_CH = 128


def _mesh():
    return plsc.VectorSubcoreMesh(
        core_axis_name="c", subcore_axis_name="s",
        num_cores=_NC, num_subcores=_NS)


_KC = 16


def _make_hop(n_tbl, n_pad, n_chunks, d):
    rows_per_tile = n_pad // _NS
    n_zero_blocks = rows_per_tile // _CH
    n_groups = n_chunks // _KC

    @functools.partial(
        pl.kernel,
        out_type=jax.ShapeDtypeStruct((_NC, n_pad, d), jnp.float32),
        mesh=_mesh(),
        scratch_types=[
            pltpu.VMEM((_KC, _CH), jnp.int32),
            pltpu.VMEM((_KC, _CH), jnp.int32),
            pltpu.VMEM((_CH, d), jnp.float32),
            pltpu.VMEM((_CH, d), jnp.float32),
            pltpu.VMEM_SHARED((n_pad, d), jnp.float32),
            pltpu.SemaphoreType.DMA,
            pltpu.SemaphoreType.DMA,
        ],
    )
    def hop(tbl_hbm, src_hbm, dst_hbm, out_hbm,
            src_v, dst_v, buf0, buf1, acc, gsem0, gsem1):
        cid = lax.axis_index("c")
        sid = lax.axis_index("s")
        wid = cid * _NS + sid

        zvec = jnp.zeros((16,), jnp.float32)

        def zbody(i, carry):
            r = i // (d // 16)
            k = i % (d // 16)
            buf0[r, pl.ds(k * 16, 16)] = zvec
            return carry

        lax.fori_loop(0, _CH * (d // 16), zbody, 0)
        base = sid * rows_per_tile
        for m in range(n_zero_blocks):
            pltpu.sync_copy(buf0, acc.at[pl.ds(base + m * _CH, _CH)])
        plsc.subcore_barrier()

        def group(g, carry):
            pltpu.sync_copy(src_hbm.at[wid, pl.ds(g * _KC, _KC)], src_v)
            pltpu.sync_copy(dst_hbm.at[wid, pl.ds(g * _KC, _KC)], dst_v)

            def body(j, carry2):
                g0 = pltpu.async_copy(tbl_hbm.at[src_v.at[2 * j]], buf0, gsem0)
                g1 = pltpu.async_copy(tbl_hbm.at[src_v.at[2 * j + 1]], buf1,
                                      gsem1)
                g0.wait()
                pltpu.sync_copy(buf0, acc.at[dst_v.at[2 * j]], add=True)
                g1.wait()
                pltpu.sync_copy(buf1, acc.at[dst_v.at[2 * j + 1]], add=True)
                return carry2

            lax.fori_loop(0, _KC // 2, body, 0)
            return carry

        lax.fori_loop(0, n_groups, group, 0)
        plsc.subcore_barrier()
        pltpu.sync_copy(acc.at[pl.ds(base, rows_per_tile)],
                        out_hbm.at[cid, pl.ds(base, rows_per_tile)])

    return hop


def _make_deg(n_pad, n_chunks, dw):
    rows_per_tile = n_pad // _NS
    n_zero_blocks = rows_per_tile // _CH

    @functools.partial(
        pl.kernel,
        out_type=jax.ShapeDtypeStruct((_NC, n_pad, dw), jnp.float32),
        mesh=_mesh(),
        scratch_types=[
            pltpu.VMEM((n_chunks, _CH), jnp.int32),
            pltpu.VMEM((_CH, dw), jnp.float32),
            pltpu.VMEM_SHARED((n_pad, dw), jnp.float32),
        ],
    )
    def deg(dst_hbm, zeros_hbm, ones_hbm, out_hbm, dst_v, ones_v, acc):
        cid = lax.axis_index("c")
        sid = lax.axis_index("s")
        wid = cid * _NS + sid
        pltpu.sync_copy(dst_hbm.at[wid], dst_v)

        pltpu.sync_copy(zeros_hbm, ones_v)
        base = sid * rows_per_tile
        for m in range(n_zero_blocks):
            pltpu.sync_copy(ones_v, acc.at[pl.ds(base + m * _CH, _CH)])
        pltpu.sync_copy(ones_hbm, ones_v)
        plsc.subcore_barrier()

        def body(j, carry):
            pltpu.sync_copy(ones_v, acc.at[dst_v.at[j]], add=True)
            return carry

        lax.fori_loop(0, n_chunks, body, 0)
        plsc.subcore_barrier()
        pltpu.sync_copy(acc.at[pl.ds(base, rows_per_tile)],
                        out_hbm.at[cid, pl.ds(base, rows_per_tile)])

    return deg


def _combine(p, degp, n, blk):
    d = p.shape[2]
    dw = degp.shape[2]

    def body(p_ref, d_ref, o_ref):
        dg = d_ref[...]
        inv = 1.0 / jnp.maximum(dg[0, :, 0] + dg[1, :, 0], 1.0)
        o_ref[...] = (p_ref[0] + p_ref[1]) * inv[:, None]

    return pl.pallas_call(
        body,
        grid=(n // blk,),
        in_specs=[
            pl.BlockSpec((2, blk, d), lambda j: (0, j, 0)),
            pl.BlockSpec((2, blk, dw), lambda j: (0, j, 0)),
        ],
        out_specs=pl.BlockSpec((blk, d), lambda j: (j, 0)),
        out_shape=jax.ShapeDtypeStruct((n, d), jnp.float32),
    )(p, degp)


def _final(x, h1, q, degp, w, b2, n, blk):
    d = x.shape[1]
    out_dim = w.shape[0]
    dw = degp.shape[2]
    dims = (((1,), (1,)), ((), ()))

    def body(x_ref, h1_ref, q_ref, d_ref, w_ref, b_ref, o_ref):
        dg = d_ref[...]
        inv = 1.0 / jnp.maximum(dg[0, :, 0] + dg[1, :, 0], 1.0)
        h2 = (q_ref[0] + q_ref[1]) * inv[:, None]
        ww = w_ref[...]
        acc = lax.dot_general(x_ref[...], ww[:, :d], dims,
                              preferred_element_type=jnp.float32)
        acc += lax.dot_general(h1_ref[...], ww[:, d:2 * d], dims,
                               preferred_element_type=jnp.float32)
        acc += lax.dot_general(h2, ww[:, 2 * d:], dims,
                               preferred_element_type=jnp.float32)
        o_ref[...] = acc + b_ref[...]

    return pl.pallas_call(
        body,
        grid=(n // blk,),
        in_specs=[
            pl.BlockSpec((blk, d), lambda j: (j, 0)),
            pl.BlockSpec((blk, d), lambda j: (j, 0)),
            pl.BlockSpec((2, blk, d), lambda j: (0, j, 0)),
            pl.BlockSpec((2, blk, dw), lambda j: (0, j, 0)),
            pl.BlockSpec(w.shape, lambda j: (0, 0)),
            pl.BlockSpec((1, out_dim), lambda j: (0, 0)),
        ],
        out_specs=pl.BlockSpec((blk, out_dim), lambda j: (j, 0)),
        out_shape=jax.ShapeDtypeStruct((n, out_dim), jnp.float32),
    )(x, h1, q, degp, w, b2)


def kernel(x, edge_index, W, b):
    n, d = x.shape
    e = edge_index.shape[1]
    out_dim = W.shape[0]

    n_chunks = -(-e // (_NW * _CH))
    n_chunks = -(-n_chunks // _KC) * _KC
    e_pad = _NW * n_chunks * _CH
    n_pad = -(-n // (_NS * _CH)) * (_NS * _CH)
    if e_pad > e and n_pad - n < 1:
        n_pad += _NS * _CH
    pad = e_pad - e
    src = edge_index[0]
    dst = edge_index[1]
    if pad:
        pad_ar = jnp.arange(pad, dtype=jnp.int32)
        src = jnp.concatenate([src, pad_ar % n])
        dst = jnp.concatenate([dst, n + pad_ar % (n_pad - n)])
    src_a = src.reshape(_NW, n_chunks, _CH)
    dst_a = dst.reshape(_NW, n_chunks, _CH)

    zeros_c = jnp.zeros((_CH, d), jnp.float32)
    ones_c = jnp.ones((_CH, d), jnp.float32)
    degp = _make_deg(n_pad, n_chunks, d)(dst_a, zeros_c, ones_c)
    hop = _make_hop(n, n_pad, n_chunks, d)
    p1 = hop(x, src_a, dst_a)
    blk = 1000 if n % 1000 == 0 else 8
    h1 = _combine(p1, degp, n, blk)
    p2 = hop(h1, src_a, dst_a)
    return _final(x, h1, p2, degp, W, b.reshape(1, out_dim), n, blk)


# speedup vs baseline: 9.2479x; 1.0909x over previous
import functools

import jax
import jax.numpy as jnp
from jax import lax
from jax.experimental import pallas as pl
from jax.experimental.pallas import tpu as pltpu
from jax.experimental.pallas import tpu_sc as plsc

_NC = 2
_NS = 16
_NW = _NC * _NS
_CH = 125
_KC = 16


def _mesh():
    return plsc.VectorSubcoreMesh(
        core_axis_name="c", subcore_axis_name="s",
        num_cores=_NC, num_subcores=_NS)


def _make_hop(n_pad, n_chunks, d):
    rows_per_tile = n_pad // _NS
    n_groups = n_chunks // _KC

    @functools.partial(
        pl.kernel,
        out_type=jax.ShapeDtypeStruct((_NC, n_pad, d), jnp.float32),
        mesh=_mesh(),
        scratch_types=[
            pltpu.VMEM((_KC, _CH), jnp.int32),
            pltpu.VMEM((_KC, _CH), jnp.int32),
            pltpu.VMEM((_CH, d), jnp.float32),
            pltpu.VMEM((_CH, d), jnp.float32),
            pltpu.VMEM_SHARED((n_pad, d), jnp.float32),
            pltpu.SemaphoreType.DMA,
            pltpu.SemaphoreType.DMA,
            pltpu.SemaphoreType.DMA,
            pltpu.SemaphoreType.DMA,
        ],
    )
    def hop(tbl_hbm, src_hbm, dst_hbm, zeros_hbm, out_hbm,
            src_v, dst_v, buf0, buf1, acc, gsem0, gsem1, ssem0, ssem1):
        cid = lax.axis_index("c")
        sid = lax.axis_index("s")
        wid = cid * _NS + sid
        base = sid * rows_per_tile
        pltpu.sync_copy(zeros_hbm, acc.at[pl.ds(base, rows_per_tile)])
        plsc.subcore_barrier()

        def group(g, carry):
            pltpu.sync_copy(src_hbm.at[wid, pl.ds(g * _KC, _KC)], src_v)
            pltpu.sync_copy(dst_hbm.at[wid, pl.ds(g * _KC, _KC)], dst_v)
            pltpu.async_copy(tbl_hbm.at[src_v.at[0]], buf0, gsem0)

            def pair(p, carry2):
                c0 = 2 * p
                c1 = c0 + 1
                pltpu.make_async_copy(
                    tbl_hbm.at[src_v.at[c0]], buf0, gsem0).wait()
                gd1 = pltpu.async_copy(tbl_hbm.at[src_v.at[c1]], buf1, gsem1)
                sd0 = pltpu.async_copy(buf0, acc.at[dst_v.at[c0]], ssem0,
                                       add=True)
                gd1.wait()
                sd1 = pltpu.async_copy(buf1, acc.at[dst_v.at[c1]], ssem1,
                                       add=True)
                sd0.wait()

                @pl.when(c1 + 1 < _KC)
                def _():
                    pltpu.async_copy(tbl_hbm.at[src_v.at[c1 + 1]], buf0, gsem0)

                sd1.wait()
                return carry2

            lax.fori_loop(0, _KC // 2, pair, 0)
            return carry

        lax.fori_loop(0, n_groups, group, 0)
        plsc.subcore_barrier()
        pltpu.sync_copy(acc.at[pl.ds(base, rows_per_tile)],
                        out_hbm.at[cid, pl.ds(base, rows_per_tile)])

    return hop


def _make_deg(n_pad, n_chunks, dw):
    rows_per_tile = n_pad // _NS

    @functools.partial(
        pl.kernel,
        out_type=jax.ShapeDtypeStruct((_NC, n_pad, dw), jnp.float32),
        mesh=_mesh(),
        scratch_types=[
            pltpu.VMEM((n_chunks, _CH), jnp.int32),
            pltpu.VMEM((_CH, dw), jnp.float32),
            pltpu.VMEM_SHARED((n_pad, dw), jnp.float32),
            pltpu.SemaphoreType.DMA,
            pltpu.SemaphoreType.DMA,
        ],
    )
    def deg(dst_hbm, zeros_hbm, ones_hbm, out_hbm,
            dst_v, ones_v, acc, ssem0, ssem1):
        cid = lax.axis_index("c")
        sid = lax.axis_index("s")
        wid = cid * _NS + sid
        pltpu.sync_copy(dst_hbm.at[wid], dst_v)

        base = sid * rows_per_tile
        pltpu.sync_copy(zeros_hbm, acc.at[pl.ds(base, rows_per_tile)])
        pltpu.sync_copy(ones_hbm, ones_v)
        plsc.subcore_barrier()

        def body(j, carry):
            sd0 = pltpu.async_copy(ones_v, acc.at[dst_v.at[2 * j]], ssem0,
                                   add=True)
            sd1 = pltpu.async_copy(ones_v, acc.at[dst_v.at[2 * j + 1]], ssem1,
                                   add=True)
            sd0.wait()
            sd1.wait()
            return carry

        lax.fori_loop(0, n_chunks // 2, body, 0)
        plsc.subcore_barrier()
        pltpu.sync_copy(acc.at[pl.ds(base, rows_per_tile)],
                        out_hbm.at[cid, pl.ds(base, rows_per_tile)])

    return deg


def _combine(p, degp, n, blk, d):
    dw = degp.shape[2]

    def body(p_ref, d_ref, h_ref, inv_ref):
        dg = d_ref[...]
        inv = 1.0 / jnp.maximum(dg[0, :, 0] + dg[1, :, 0], 1.0)
        h_ref[...] = (p_ref[0] + p_ref[1]) * inv[:, None]
        inv_ref[...] = jnp.broadcast_to(inv[:, None], (blk, d))

    return pl.pallas_call(
        body,
        grid=(n // blk,),
        in_specs=[pl.BlockSpec((2, blk, d), lambda j: (0, j, 0)),
                  pl.BlockSpec((2, blk, dw), lambda j: (0, j, 0))],
        out_specs=[pl.BlockSpec((blk, d), lambda j: (j, 0)),
                   pl.BlockSpec((blk, d), lambda j: (j, 0))],
        out_shape=[jax.ShapeDtypeStruct((n, d), jnp.float32),
                   jax.ShapeDtypeStruct((n, d), jnp.float32)],
    )(p, degp)


def _final(x, h1, q, inv, w, b2, n, blk):
    d = x.shape[1]
    out_dim = w.shape[0]
    dims = (((1,), (1,)), ((), ()))

    def body(x_ref, h1_ref, q_ref, inv_ref, w_ref, b_ref, o_ref):
        h2 = (q_ref[0] + q_ref[1]) * inv_ref[...]
        ww = w_ref[...]
        acc = lax.dot_general(x_ref[...], ww[:, :d], dims,
                              preferred_element_type=jnp.float32)
        acc += lax.dot_general(h1_ref[...], ww[:, d:2 * d], dims,
                               preferred_element_type=jnp.float32)
        acc += lax.dot_general(h2, ww[:, 2 * d:], dims,
                               preferred_element_type=jnp.float32)
        o_ref[...] = acc + b_ref[...]

    return pl.pallas_call(
        body,
        grid=(n // blk,),
        in_specs=[
            pl.BlockSpec((blk, d), lambda j: (j, 0)),
            pl.BlockSpec((blk, d), lambda j: (j, 0)),
            pl.BlockSpec((2, blk, d), lambda j: (0, j, 0)),
            pl.BlockSpec((blk, d), lambda j: (j, 0)),
            pl.BlockSpec(w.shape, lambda j: (0, 0)),
            pl.BlockSpec((1, out_dim), lambda j: (0, 0)),
        ],
        out_specs=pl.BlockSpec((blk, out_dim), lambda j: (j, 0)),
        out_shape=jax.ShapeDtypeStruct((n, out_dim), jnp.float32),
    )(x, h1, q, inv, w, b2)


def kernel(x, edge_index, W, b):
    n, d = x.shape
    e = edge_index.shape[1]
    out_dim = W.shape[0]

    n_chunks = -(-e // (_NW * _CH))
    n_chunks = -(-n_chunks // _KC) * _KC
    e_pad = _NW * n_chunks * _CH
    n_pad = -(-n // (_NS * 128)) * (_NS * 128)
    if e_pad > e and n_pad - n < 1:
        n_pad += _NS * 128
    pad = e_pad - e
    src = edge_index[0]
    dst = edge_index[1]
    if pad:
        pad_ar = jnp.arange(pad, dtype=jnp.int32)
        src = jnp.concatenate([src, pad_ar % n])
        dst = jnp.concatenate([dst, n + pad_ar % (n_pad - n)])
    src_a = src.reshape(_NW, n_chunks, _CH)
    dst_a = dst.reshape(_NW, n_chunks, _CH)

    rows_per_tile = n_pad // _NS
    dw = 128
    degp = _make_deg(n_pad, n_chunks, dw)(
        dst_a, jnp.zeros((rows_per_tile, dw), jnp.float32),
        jnp.ones((_CH, dw), jnp.float32))
    hop = _make_hop(n_pad, n_chunks, d)
    zeros_d = jnp.zeros((rows_per_tile, d), jnp.float32)
    p1 = hop(x, src_a, dst_a, zeros_d)
    blk = 1000 if n % 1000 == 0 else 8
    h1, inv = _combine(p1, degp, n, blk, d)
    p2 = hop(h1, src_a, dst_a, zeros_d)
    return _final(x, h1, p2, inv, W, b.reshape(1, out_dim), n, blk)


# speedup vs baseline: 10.8939x; 1.1780x over previous
import functools

import jax
import jax.numpy as jnp
from jax import lax
from jax.experimental import pallas as pl
from jax.experimental.pallas import tpu as pltpu
from jax.experimental.pallas import tpu_sc as plsc

_NC = 2
_NS = 16
_NW = _NC * _NS
_CH = 125
_KC = 16


def _mesh():
    return plsc.VectorSubcoreMesh(
        core_axis_name="c", subcore_axis_name="s",
        num_cores=_NC, num_subcores=_NS)


def _make_hop(n_pad, n_chunks, d):
    rows_per_tile = n_pad // _NS
    n_groups = n_chunks // _KC

    @functools.partial(
        pl.kernel,
        out_type=jax.ShapeDtypeStruct((_NC, n_pad, d), jnp.float32),
        mesh=_mesh(),
        scratch_types=[
            pltpu.VMEM((_KC, _CH), jnp.int32),
            pltpu.VMEM((_KC, _CH), jnp.int32),
            pltpu.VMEM((_CH, d), jnp.float32),
            pltpu.VMEM((_CH, d), jnp.float32),
            pltpu.VMEM_SHARED((n_pad, d), jnp.float32),
            pltpu.SemaphoreType.DMA,
            pltpu.SemaphoreType.DMA,
            pltpu.SemaphoreType.DMA,
            pltpu.SemaphoreType.DMA,
        ],
    )
    def hop(tbl_hbm, src_hbm, dst_hbm, zeros_hbm, out_hbm,
            src_v, dst_v, buf0, buf1, acc, gsem0, gsem1, ssem0, ssem1):
        cid = lax.axis_index("c")
        sid = lax.axis_index("s")
        wid = cid * _NS + sid
        base = sid * rows_per_tile
        pltpu.sync_copy(zeros_hbm, acc.at[pl.ds(base, rows_per_tile)])
        plsc.subcore_barrier()

        def group(g, carry):
            pltpu.sync_copy(src_hbm.at[wid, pl.ds(g * _KC, _KC)], src_v)
            pltpu.sync_copy(dst_hbm.at[wid, pl.ds(g * _KC, _KC)], dst_v)
            pltpu.async_copy(tbl_hbm.at[src_v.at[0]], buf0, gsem0)

            def pair(p, carry2):
                c0 = 2 * p
                c1 = c0 + 1
                pltpu.make_async_copy(
                    tbl_hbm.at[src_v.at[c0]], buf0, gsem0).wait()
                gd1 = pltpu.async_copy(tbl_hbm.at[src_v.at[c1]], buf1, gsem1)
                sd0 = pltpu.async_copy(buf0, acc.at[dst_v.at[c0]], ssem0,
                                       add=True)
                gd1.wait()
                sd1 = pltpu.async_copy(buf1, acc.at[dst_v.at[c1]], ssem1,
                                       add=True)
                sd0.wait()

                @pl.when(c1 + 1 < _KC)
                def _():
                    pltpu.async_copy(tbl_hbm.at[src_v.at[c1 + 1]], buf0, gsem0)

                sd1.wait()
                return carry2

            lax.fori_loop(0, _KC // 2, pair, 0)
            return carry

        lax.fori_loop(0, n_groups, group, 0)
        plsc.subcore_barrier()
        pltpu.sync_copy(acc.at[pl.ds(base, rows_per_tile)],
                        out_hbm.at[cid, pl.ds(base, rows_per_tile)])

    return hop


def _make_deg(n_pad, e_tile):
    n_vec = e_tile // 16

    @functools.partial(
        pl.kernel,
        out_type=jax.ShapeDtypeStruct((_NW, n_pad), jnp.float32),
        mesh=_mesh(),
        scratch_types=[
            pltpu.VMEM((e_tile,), jnp.int32),
            pltpu.VMEM((n_pad,), jnp.float32),
        ],
        compiler_params=pltpu.CompilerParams(needs_layout_passes=False),
    )
    def deg(dst_hbm, zeros_hbm, out_hbm, dst_v, hist):
        cid = lax.axis_index("c")
        sid = lax.axis_index("s")
        wid = cid * _NS + sid
        pltpu.sync_copy(dst_hbm.at[wid], dst_v)
        pltpu.sync_copy(zeros_hbm, hist)
        ones_v = jnp.full((16,), 1.0, jnp.float32)

        def body(v, carry):
            idx = dst_v[pl.ds(v * 16, 16)]
            plsc.addupdate_scatter(hist, [idx], ones_v)
            return carry

        lax.fori_loop(0, n_vec, body, 0)
        pltpu.sync_copy(hist, out_hbm.at[wid])

    return deg


def _combine(p, degt, n, blk, d):
    nw = degt.shape[1]

    def body(p_ref, d_ref, h_ref, inv_ref):
        deg = jnp.sum(d_ref[...], axis=1)
        inv = 1.0 / jnp.maximum(deg, 1.0)
        h_ref[...] = (p_ref[0] + p_ref[1]) * inv[:, None]
        inv_ref[...] = jnp.broadcast_to(inv[:, None], (blk, d))

    return pl.pallas_call(
        body,
        grid=(n // blk,),
        in_specs=[pl.BlockSpec((2, blk, d), lambda j: (0, j, 0)),
                  pl.BlockSpec((blk, nw), lambda j: (j, 0))],
        out_specs=[pl.BlockSpec((blk, d), lambda j: (j, 0)),
                   pl.BlockSpec((blk, d), lambda j: (j, 0))],
        out_shape=[jax.ShapeDtypeStruct((n, d), jnp.float32),
                   jax.ShapeDtypeStruct((n, d), jnp.float32)],
    )(p, degt)


def _final(x, h1, q, inv, w, b2, n, blk):
    d = x.shape[1]
    out_dim = w.shape[0]
    dims = (((1,), (1,)), ((), ()))

    def body(x_ref, h1_ref, q_ref, inv_ref, w_ref, b_ref, o_ref):
        h2 = (q_ref[0] + q_ref[1]) * inv_ref[...]
        ww = w_ref[...]
        acc = lax.dot_general(x_ref[...], ww[:, :d], dims,
                              preferred_element_type=jnp.float32)
        acc += lax.dot_general(h1_ref[...], ww[:, d:2 * d], dims,
                               preferred_element_type=jnp.float32)
        acc += lax.dot_general(h2, ww[:, 2 * d:], dims,
                               preferred_element_type=jnp.float32)
        o_ref[...] = acc + b_ref[...]

    return pl.pallas_call(
        body,
        grid=(n // blk,),
        in_specs=[
            pl.BlockSpec((blk, d), lambda j: (j, 0)),
            pl.BlockSpec((blk, d), lambda j: (j, 0)),
            pl.BlockSpec((2, blk, d), lambda j: (0, j, 0)),
            pl.BlockSpec((blk, d), lambda j: (j, 0)),
            pl.BlockSpec(w.shape, lambda j: (0, 0)),
            pl.BlockSpec((1, out_dim), lambda j: (0, 0)),
        ],
        out_specs=pl.BlockSpec((blk, out_dim), lambda j: (j, 0)),
        out_shape=jax.ShapeDtypeStruct((n, out_dim), jnp.float32),
    )(x, h1, q, inv, w, b2)


def kernel(x, edge_index, W, b):
    n, d = x.shape
    e = edge_index.shape[1]
    out_dim = W.shape[0]

    n_chunks = -(-e // (_NW * _CH))
    n_chunks = -(-n_chunks // _KC) * _KC
    e_pad = _NW * n_chunks * _CH
    n_pad = -(-n // (_NS * 128)) * (_NS * 128)
    if e_pad > e and n_pad - n < 1:
        n_pad += _NS * 128
    pad = e_pad - e
    src = edge_index[0]
    dst = edge_index[1]
    if pad:
        pad_ar = jnp.arange(pad, dtype=jnp.int32)
        src = jnp.concatenate([src, pad_ar % n])
        dst = jnp.concatenate([dst, n + pad_ar % (n_pad - n)])
    src_a = src.reshape(_NW, n_chunks, _CH)
    dst_a = dst.reshape(_NW, n_chunks, _CH)

    rows_per_tile = n_pad // _NS
    e_tile = n_chunks * _CH
    degp = _make_deg(n_pad, e_tile)(
        dst.reshape(_NW, e_tile), jnp.zeros((n_pad,), jnp.float32))
    hop = _make_hop(n_pad, n_chunks, d)
    zeros_d = jnp.zeros((rows_per_tile, d), jnp.float32)
    p1 = hop(x, src_a, dst_a, zeros_d)
    blk = 1000 if n % 1000 == 0 else 8
    blk2 = 1024 if n_pad % 1024 == 0 else 128
    h1, inv = _combine(p1, degp.T, n_pad, blk2, d)
    p2 = hop(h1, src_a, dst_a, zeros_d)
    return _final(x, h1, p2, inv, W, b.reshape(1, out_dim), n, blk)


# speedup vs baseline: 10.9812x; 1.0080x over previous
import functools

import jax
import jax.numpy as jnp
from jax import lax
from jax.experimental import pallas as pl
from jax.experimental.pallas import tpu as pltpu
from jax.experimental.pallas import tpu_sc as plsc

_NC = 2
_NS = 16
_NW = _NC * _NS
_CH = 125
_KC = 16


def _mesh():
    return plsc.VectorSubcoreMesh(
        core_axis_name="c", subcore_axis_name="s",
        num_cores=_NC, num_subcores=_NS)


def _make_hop(n_pad, n_chunks, d):
    rows_per_tile = n_pad // _NS
    n_groups = n_chunks // _KC

    @functools.partial(
        pl.kernel,
        out_type=jax.ShapeDtypeStruct((_NC, n_pad, d), jnp.float32),
        mesh=_mesh(),
        scratch_types=[
            pltpu.VMEM((_KC, _CH), jnp.int32),
            pltpu.VMEM((_KC, _CH), jnp.int32),
            pltpu.VMEM((_CH, d), jnp.float32),
            pltpu.VMEM((_CH, d), jnp.float32),
            pltpu.VMEM_SHARED((n_pad, d), jnp.float32),
            pltpu.SemaphoreType.DMA,
            pltpu.SemaphoreType.DMA,
            pltpu.SemaphoreType.DMA,
            pltpu.SemaphoreType.DMA,
        ],
    )
    def hop(tbl_hbm, src_hbm, dst_hbm, zeros_hbm, out_hbm,
            src_v, dst_v, buf0, buf1, acc, gsem0, gsem1, ssem0, ssem1):
        cid = lax.axis_index("c")
        sid = lax.axis_index("s")
        wid = cid * _NS + sid
        base = sid * rows_per_tile
        pltpu.sync_copy(zeros_hbm, acc.at[pl.ds(base, rows_per_tile)])
        plsc.subcore_barrier()

        def group(g, carry):
            pltpu.sync_copy(src_hbm.at[wid, pl.ds(g * _KC, _KC)], src_v)
            pltpu.sync_copy(dst_hbm.at[wid, pl.ds(g * _KC, _KC)], dst_v)
            pltpu.async_copy(tbl_hbm.at[src_v.at[0]], buf0, gsem0)

            def pair(p, carry2):
                c0 = 2 * p
                c1 = c0 + 1
                pltpu.make_async_copy(
                    tbl_hbm.at[src_v.at[c0]], buf0, gsem0).wait()
                gd1 = pltpu.async_copy(tbl_hbm.at[src_v.at[c1]], buf1, gsem1)
                sd0 = pltpu.async_copy(buf0, acc.at[dst_v.at[c0]], ssem0,
                                       add=True)
                gd1.wait()
                sd1 = pltpu.async_copy(buf1, acc.at[dst_v.at[c1]], ssem1,
                                       add=True)
                sd0.wait()

                @pl.when(c1 + 1 < _KC)
                def _():
                    pltpu.async_copy(tbl_hbm.at[src_v.at[c1 + 1]], buf0, gsem0)

                sd1.wait()
                return carry2

            lax.fori_loop(0, _KC // 2, pair, 0)
            return carry

        lax.fori_loop(0, n_groups, group, 0)
        plsc.subcore_barrier()
        pltpu.sync_copy(acc.at[pl.ds(base, rows_per_tile)],
                        out_hbm.at[cid, pl.ds(base, rows_per_tile)])

    return hop


def _make_deg(n_pad, e_tile):
    n_vec = e_tile // 16

    @functools.partial(
        pl.kernel,
        out_type=jax.ShapeDtypeStruct((_NW, n_pad), jnp.float32),
        mesh=_mesh(),
        scratch_types=[
            pltpu.VMEM((e_tile,), jnp.int32),
            pltpu.VMEM((n_pad,), jnp.float32),
        ],
        compiler_params=pltpu.CompilerParams(needs_layout_passes=False),
    )
    def deg(dst_hbm, zeros_hbm, out_hbm, dst_v, hist):
        cid = lax.axis_index("c")
        sid = lax.axis_index("s")
        wid = cid * _NS + sid
        pltpu.sync_copy(dst_hbm.at[wid], dst_v)
        pltpu.sync_copy(zeros_hbm, hist)
        ones_v = jnp.full((16,), 1.0, jnp.float32)

        def body(v, carry):
            idx = dst_v[pl.ds(v * 16, 16)]
            plsc.addupdate_scatter(hist, [idx], ones_v)
            return carry

        lax.fori_loop(0, n_vec, body, 0)
        pltpu.sync_copy(hist, out_hbm.at[wid])

    return deg


def _combine(p, degt, n, blk, d):
    nw = degt.shape[1]

    def body(p_ref, d_ref, h_ref):
        deg = jnp.sum(d_ref[...], axis=1)
        inv = 1.0 / jnp.maximum(deg, 1.0)
        h_ref[...] = (p_ref[0] + p_ref[1]) * inv[:, None]

    return pl.pallas_call(
        body,
        grid=(n // blk,),
        in_specs=[pl.BlockSpec((2, blk, d), lambda j: (0, j, 0)),
                  pl.BlockSpec((blk, nw), lambda j: (j, 0))],
        out_specs=pl.BlockSpec((blk, d), lambda j: (j, 0)),
        out_shape=jax.ShapeDtypeStruct((n, d), jnp.float32),
    )(p, degt)


def _final(x, h1, q, degt, w, b2, n, blk):
    d = x.shape[1]
    out_dim = w.shape[0]
    nw = degt.shape[1]
    dims = (((1,), (1,)), ((), ()))

    def body(x_ref, h1_ref, q_ref, d_ref, w_ref, b_ref, o_ref):
        deg = jnp.sum(d_ref[...], axis=1)
        inv = 1.0 / jnp.maximum(deg, 1.0)
        h2 = (q_ref[0] + q_ref[1]) * inv[:, None]
        ww = w_ref[...]
        acc = lax.dot_general(x_ref[...], ww[:, :d], dims,
                              preferred_element_type=jnp.float32)
        acc += lax.dot_general(h1_ref[...], ww[:, d:2 * d], dims,
                               preferred_element_type=jnp.float32)
        acc += lax.dot_general(h2, ww[:, 2 * d:], dims,
                               preferred_element_type=jnp.float32)
        o_ref[...] = acc + b_ref[...]

    return pl.pallas_call(
        body,
        grid=(n // blk,),
        in_specs=[
            pl.BlockSpec((blk, d), lambda j: (j, 0)),
            pl.BlockSpec((blk, d), lambda j: (j, 0)),
            pl.BlockSpec((2, blk, d), lambda j: (0, j, 0)),
            pl.BlockSpec((blk, nw), lambda j: (j, 0)),
            pl.BlockSpec(w.shape, lambda j: (0, 0)),
            pl.BlockSpec((1, out_dim), lambda j: (0, 0)),
        ],
        out_specs=pl.BlockSpec((blk, out_dim), lambda j: (j, 0)),
        out_shape=jax.ShapeDtypeStruct((n, out_dim), jnp.float32),
    )(x, h1, q, degt, w, b2)


def kernel(x, edge_index, W, b):
    n, d = x.shape
    e = edge_index.shape[1]
    out_dim = W.shape[0]

    n_chunks = -(-e // (_NW * _CH))
    n_chunks = -(-n_chunks // _KC) * _KC
    e_pad = _NW * n_chunks * _CH
    n_pad = -(-n // (_NS * 128)) * (_NS * 128)
    if e_pad > e and n_pad - n < 1:
        n_pad += _NS * 128
    pad = e_pad - e
    src = edge_index[0]
    dst = edge_index[1]
    if pad:
        pad_ar = jnp.arange(pad, dtype=jnp.int32)
        src = jnp.concatenate([src, pad_ar % n])
        dst = jnp.concatenate([dst, n + pad_ar % (n_pad - n)])
    src_a = src.reshape(_NW, n_chunks, _CH)
    dst_a = dst.reshape(_NW, n_chunks, _CH)

    rows_per_tile = n_pad // _NS
    e_tile = n_chunks * _CH
    degp = _make_deg(n_pad, e_tile)(
        dst.reshape(_NW, e_tile), jnp.zeros((n_pad,), jnp.float32))
    hop = _make_hop(n_pad, n_chunks, d)
    zeros_d = jnp.zeros((rows_per_tile, d), jnp.float32)
    p1 = hop(x, src_a, dst_a, zeros_d)
    blk = 1000 if n % 1000 == 0 else 8
    blk2 = 1024 if n_pad % 1024 == 0 else 128
    degt = degp.T
    h1 = _combine(p1, degt, n_pad, blk2, d)
    p2 = hop(h1, src_a, dst_a, zeros_d)
    return _final(x, h1, p2, degt, W, b.reshape(1, out_dim), n, blk)
